# Initial kernel scaffold; baseline (speedup 1.0000x reference)
import jax
import jax.numpy as jnp
from jax.experimental import pallas as pl


def kernel(ids, char_embed, W, b):
    raise NotImplementedError("write your pallas kernel here")


# speedup vs baseline: 68.0085x; 68.0085x over previous
import jax
import jax.numpy as jnp
from jax import lax
from jax.experimental import pallas as pl
from jax.experimental.pallas import tpu as pltpu
from jax.experimental.pallas import tpu_sc as plsc

B = 16384
L = 200
V = 256
D = 128

NUM_CORES = 2
NUM_SUBCORES = 16
NW = NUM_CORES * NUM_SUBCORES
LANES = 16

ROWS_PER_W = B // NW
CHUNK = 64
N_CHUNKS = ROWS_PER_W // CHUNK
GROUPS = CHUNK // LANES


def _hist_body(ids_hbm, counts_hbm, ids_v, counts_v):
  wid = lax.axis_index("s") * NUM_CORES + lax.axis_index("c")
  lane = lax.iota(jnp.int32, LANES)
  ones = jnp.ones((LANES,), jnp.float32)
  zeros = jnp.zeros((LANES,), jnp.float32)

  def chunk_body(ci, carry):
    row0 = wid * ROWS_PER_W + ci * CHUNK
    pltpu.sync_copy(ids_hbm.at[pl.ds(row0 * L, CHUNK * L)], ids_v)

    def zero_body(i, c):
      counts_v[pl.ds(i * LANES, LANES)] = zeros
      return c

    lax.fori_loop(0, CHUNK * V // LANES, zero_body, 0)

    for g in range(GROUPS):
      load_base = lane * L + g * LANES * L
      bin_base = lane * V + g * LANES * V

      def l_body(l, c):
        ids_vec = plsc.load_gather(ids_v, [load_base + l])
        plsc.addupdate_scatter(counts_v, [bin_base + ids_vec], ones)
        return c

      lax.fori_loop(0, L, l_body, 0)

    pltpu.sync_copy(counts_v, counts_hbm.at[pl.ds(row0 * V, CHUNK * V)])
    return carry

  lax.fori_loop(0, N_CHUNKS, chunk_body, 0)


_hist = pl.kernel(
    _hist_body,
    out_type=jax.ShapeDtypeStruct((B * V,), jnp.float32),
    mesh=plsc.VectorSubcoreMesh(core_axis_name="c", subcore_axis_name="s",
                                num_cores=NUM_CORES, num_subcores=NUM_SUBCORES),
    scratch_types=[
        pltpu.VMEM((CHUNK * L,), jnp.int32),
        pltpu.VMEM((CHUNK * V,), jnp.float32),
    ],
    compiler_params=pltpu.CompilerParams(needs_layout_passes=False),
)

BR = 512


def _mlp_body(counts_ref, e_ref, wt_ref, b_ref, out_ref):
  pooled = jnp.dot(counts_ref[...], e_ref[...],
                   preferred_element_type=jnp.float32) * (1.0 / L)
  y = jnp.dot(pooled, wt_ref[...], preferred_element_type=jnp.float32)
  y = y + b_ref[...]
  out_ref[...] = 0.5 * y * (1.0 + lax.erf(y * (2.0 ** -0.5)))


_mlp = pl.pallas_call(
    _mlp_body,
    grid=(B // BR,),
    in_specs=[
        pl.BlockSpec((BR, V), lambda i: (i, 0)),
        pl.BlockSpec((V, D), lambda i: (0, 0)),
        pl.BlockSpec((D, D), lambda i: (0, 0)),
        pl.BlockSpec((1, D), lambda i: (0, 0)),
    ],
    out_specs=pl.BlockSpec((BR, D), lambda i: (i, 0)),
    out_shape=jax.ShapeDtypeStruct((B, D), jnp.float32),
)


def kernel(ids, char_embed, W, b):
  ids_flat = ids.astype(jnp.int32).reshape(B * L)
  counts = _hist(ids_flat).reshape(B, V)
  return _mlp(counts, char_embed, W.T, b.reshape(1, D))


# speedup vs baseline: 79.4298x; 1.1679x over previous
import jax
import jax.numpy as jnp
from jax import lax
from jax.experimental import pallas as pl
from jax.experimental.pallas import tpu as pltpu
from jax.experimental.pallas import tpu_sc as plsc

B = 16384
L = 200
V = 256
D = 128

NUM_CORES = 2
NUM_SUBCORES = 16
NW = NUM_CORES * NUM_SUBCORES
LANES = 16

ROWS_PER_W = B // NW
CHUNK = 64
N_CHUNKS = ROWS_PER_W // CHUNK
GROUPS = CHUNK // LANES


def _hist_body(ids_hbm, counts_hbm, ids_v, counts_v):
  wid = lax.axis_index("s") * NUM_CORES + lax.axis_index("c")
  lane = lax.iota(jnp.int32, LANES)
  ones = jnp.ones((LANES,), jnp.float32)
  zeros = jnp.zeros((LANES,), jnp.float32)

  load_bases = [lane * L + g * LANES * L for g in range(GROUPS)]
  bin_bases = [lane * V + g * LANES * V for g in range(GROUPS)]

  ZU = 16
  LU = 8

  def chunk_body(ci, carry):
    row0 = wid * ROWS_PER_W + ci * CHUNK
    pltpu.sync_copy(ids_hbm.at[pl.ds(row0 * L, CHUNK * L)], ids_v)

    def zero_body(i, c):
      for u in range(ZU):
        counts_v[pl.ds((i * ZU + u) * LANES, LANES)] = zeros
      return c

    lax.fori_loop(0, CHUNK * V // (LANES * ZU), zero_body, 0)

    def l_body(lo, c):
      for u in range(LU):
        l = lo * LU + u
        for g in range(GROUPS):
          ids_vec = plsc.load_gather(ids_v, [load_bases[g] + l])
          plsc.addupdate_scatter(counts_v, [bin_bases[g] + ids_vec], ones)
      return c

    lax.fori_loop(0, L // LU, l_body, 0)

    pltpu.sync_copy(counts_v, counts_hbm.at[pl.ds(row0 * V, CHUNK * V)])
    return carry

  lax.fori_loop(0, N_CHUNKS, chunk_body, 0)


_hist = pl.kernel(
    _hist_body,
    out_type=jax.ShapeDtypeStruct((B * V,), jnp.float32),
    mesh=plsc.VectorSubcoreMesh(core_axis_name="c", subcore_axis_name="s",
                                num_cores=NUM_CORES, num_subcores=NUM_SUBCORES),
    scratch_types=[
        pltpu.VMEM((CHUNK * L,), jnp.int32),
        pltpu.VMEM((CHUNK * V,), jnp.float32),
    ],
    compiler_params=pltpu.CompilerParams(needs_layout_passes=False),
)

BR = 512


def _mlp_body(counts_ref, e_ref, wt_ref, b_ref, out_ref):
  pooled = jnp.dot(counts_ref[...], e_ref[...],
                   preferred_element_type=jnp.float32) * (1.0 / L)
  y = jnp.dot(pooled, wt_ref[...], preferred_element_type=jnp.float32)
  y = y + b_ref[...]
  out_ref[...] = 0.5 * y * (1.0 + lax.erf(y * (2.0 ** -0.5)))


_mlp = pl.pallas_call(
    _mlp_body,
    grid=(B // BR,),
    in_specs=[
        pl.BlockSpec((BR, V), lambda i: (i, 0)),
        pl.BlockSpec((V, D), lambda i: (0, 0)),
        pl.BlockSpec((D, D), lambda i: (0, 0)),
        pl.BlockSpec((1, D), lambda i: (0, 0)),
    ],
    out_specs=pl.BlockSpec((BR, D), lambda i: (i, 0)),
    out_shape=jax.ShapeDtypeStruct((B, D), jnp.float32),
)


def kernel(ids, char_embed, W, b):
  ids_flat = ids.astype(jnp.int32).reshape(B * L)
  counts = _hist(ids_flat).reshape(B, V)
  return _mlp(counts, char_embed, W.T, b.reshape(1, D))


# speedup vs baseline: 99.5597x; 1.2534x over previous
import jax
import jax.numpy as jnp
from jax import lax
from jax.experimental import pallas as pl
from jax.experimental.pallas import tpu as pltpu
from jax.experimental.pallas import tpu_sc as plsc

B = 16384
L = 200
V = 256
D = 128

NUM_CORES = 2
NUM_SUBCORES = 16
NW = NUM_CORES * NUM_SUBCORES
LANES = 16

ROWS_PER_W = B // NW
CHUNK = 64
N_CHUNKS = ROWS_PER_W // CHUNK
GROUPS = CHUNK // LANES


def _hist_body(ids_hbm, counts_hbm, ids_v, counts_v):
  wid = lax.axis_index("s") * NUM_CORES + lax.axis_index("c")
  lane = lax.iota(jnp.int32, LANES)
  ones = jnp.ones((LANES,), jnp.float32)
  zeros = jnp.zeros((LANES,), jnp.float32)

  load_bases = [lane * L + g * LANES * L for g in range(GROUPS)]
  bin_bases = [lane * V + g * LANES * V for g in range(GROUPS)]

  ZU = 16
  LU = 8

  def chunk_body(ci, carry):
    row0 = wid * ROWS_PER_W + ci * CHUNK
    pltpu.sync_copy(ids_hbm.at[pl.ds(row0 * L, CHUNK * L)], ids_v)

    def zero_body(i, c):
      for u in range(ZU):
        counts_v[pl.ds((i * ZU + u) * LANES, LANES)] = zeros
      return c

    lax.fori_loop(0, CHUNK * V // (LANES * ZU), zero_body, 0)

    def l_body(lo, c):
      for u in range(0, LU, 2):
        l = lo * LU + u
        vecs = [plsc.load_gather(ids_v, [load_bases[g] + (l + du)])
                for du in range(2) for g in range(GROUPS)]
        idxs = [bin_bases[g] + vecs[du * GROUPS + g]
                for du in range(2) for g in range(GROUPS)]
        for i in range(2 * GROUPS):
          plsc.addupdate_scatter(counts_v, [idxs[i]], ones)
      return c

    lax.fori_loop(0, L // LU, l_body, 0)

    pltpu.sync_copy(counts_v, counts_hbm.at[pl.ds(row0 * V, CHUNK * V)])
    return carry

  lax.fori_loop(0, N_CHUNKS, chunk_body, 0)


_hist = pl.kernel(
    _hist_body,
    out_type=jax.ShapeDtypeStruct((B * V,), jnp.float32),
    mesh=plsc.VectorSubcoreMesh(core_axis_name="c", subcore_axis_name="s",
                                num_cores=NUM_CORES, num_subcores=NUM_SUBCORES),
    scratch_types=[
        pltpu.VMEM((CHUNK * L,), jnp.int32),
        pltpu.VMEM((CHUNK * V,), jnp.float32),
    ],
    compiler_params=pltpu.CompilerParams(needs_layout_passes=False),
)

BR = 512


def _mlp_body(counts_ref, e_ref, wt_ref, b_ref, out_ref):
  pooled = jnp.dot(counts_ref[...], e_ref[...],
                   preferred_element_type=jnp.float32) * (1.0 / L)
  y = jnp.dot(pooled, wt_ref[...], preferred_element_type=jnp.float32)
  y = y + b_ref[...]
  out_ref[...] = 0.5 * y * (1.0 + lax.erf(y * (2.0 ** -0.5)))


_mlp = pl.pallas_call(
    _mlp_body,
    grid=(B // BR,),
    in_specs=[
        pl.BlockSpec((BR, V), lambda i: (i, 0)),
        pl.BlockSpec((V, D), lambda i: (0, 0)),
        pl.BlockSpec((D, D), lambda i: (0, 0)),
        pl.BlockSpec((1, D), lambda i: (0, 0)),
    ],
    out_specs=pl.BlockSpec((BR, D), lambda i: (i, 0)),
    out_shape=jax.ShapeDtypeStruct((B, D), jnp.float32),
)


def kernel(ids, char_embed, W, b):
  ids_flat = ids.astype(jnp.int32).reshape(B * L)
  counts = _hist(ids_flat).reshape(B, V)
  return _mlp(counts, char_embed, W.T, b.reshape(1, D))


# speedup vs baseline: 109.8472x; 1.1033x over previous
import jax
import jax.numpy as jnp
from jax import lax
from jax.experimental import pallas as pl
from jax.experimental.pallas import tpu as pltpu
from jax.experimental.pallas import tpu_sc as plsc

B = 16384
L = 200
V = 256
D = 128

NUM_CORES = 2
NUM_SUBCORES = 16
NW = NUM_CORES * NUM_SUBCORES
LANES = 16

ROWS_PER_W = B // NW
CHUNK = 64
N_CHUNKS = ROWS_PER_W // CHUNK
GROUPS = CHUNK // LANES


def _hist_body(ids_hbm, counts_hbm, ids_v0, ids_v1, counts_v0, counts_v1,
               sem_in0, sem_in1, sem_out0, sem_out1):
  wid = lax.axis_index("s") * NUM_CORES + lax.axis_index("c")
  lane = lax.iota(jnp.int32, LANES)
  ones = jnp.ones((LANES,), jnp.float32)
  zeros = jnp.zeros((LANES,), jnp.float32)

  load_bases = [lane * L + g * LANES * L for g in range(GROUPS)]
  bin_bases = [lane * V + g * LANES * V for g in range(GROUPS)]

  ids_vs = (ids_v0, ids_v1)
  counts_vs = (counts_v0, counts_v1)
  sems_in = (sem_in0, sem_in1)
  sems_out = (sem_out0, sem_out1)

  ZU = 16
  LU = 8

  def cp_in(ci, s):
    row0 = wid * ROWS_PER_W + ci * CHUNK
    return pltpu.make_async_copy(
        ids_hbm.at[pl.ds(row0 * L, CHUNK * L)], ids_vs[s], sems_in[s])

  def cp_out(ci, s):
    row0 = wid * ROWS_PER_W + ci * CHUNK
    return pltpu.make_async_copy(
        counts_vs[s], counts_hbm.at[pl.ds(row0 * V, CHUNK * V)], sems_out[s])

  def histogram(ids_v, counts_v):
    def zero_body(i, c):
      for u in range(ZU):
        counts_v[pl.ds((i * ZU + u) * LANES, LANES)] = zeros
      return c

    lax.fori_loop(0, CHUNK * V // (LANES * ZU), zero_body, 0)

    def l_body(lo, c):
      for u in range(0, LU, 2):
        l = lo * LU + u
        vecs = [plsc.load_gather(ids_v, [load_bases[g] + (l + du)])
                for du in range(2) for g in range(GROUPS)]
        idxs = [bin_bases[g] + vecs[du * GROUPS + g]
                for du in range(2) for g in range(GROUPS)]
        for i in range(2 * GROUPS):
          plsc.addupdate_scatter(counts_v, [idxs[i]], ones)
      return c

    lax.fori_loop(0, L // LU, l_body, 0)

  cp_in(0, 0).start()
  cp_in(1, 1).start()

  def pair_body(p, carry):
    for k in range(2):
      ci = 2 * p + k
      s = k

      @pl.when(p > 0)
      def _():
        cp_out(ci - 2, s).wait()

      cp_in(ci, s).wait()
      histogram(ids_vs[s], counts_vs[s])

      @pl.when(p < N_CHUNKS // 2 - 1)
      def _():
        cp_in(ci + 2, s).start()

      cp_out(ci, s).start()
    return carry

  lax.fori_loop(0, N_CHUNKS // 2, pair_body, 0)
  cp_out(N_CHUNKS - 2, 0).wait()
  cp_out(N_CHUNKS - 1, 1).wait()


_hist = pl.kernel(
    _hist_body,
    out_type=jax.ShapeDtypeStruct((B * V,), jnp.float32),
    mesh=plsc.VectorSubcoreMesh(core_axis_name="c", subcore_axis_name="s",
                                num_cores=NUM_CORES, num_subcores=NUM_SUBCORES),
    scratch_types=[
        pltpu.VMEM((CHUNK * L,), jnp.int32),
        pltpu.VMEM((CHUNK * L,), jnp.int32),
        pltpu.VMEM((CHUNK * V,), jnp.float32),
        pltpu.VMEM((CHUNK * V,), jnp.float32),
        pltpu.SemaphoreType.DMA,
        pltpu.SemaphoreType.DMA,
        pltpu.SemaphoreType.DMA,
        pltpu.SemaphoreType.DMA,
    ],
    compiler_params=pltpu.CompilerParams(needs_layout_passes=False),
)

BR = 512


def _mlp_body(counts_ref, e_ref, wt_ref, b_ref, out_ref):
  pooled = jnp.dot(counts_ref[...], e_ref[...],
                   preferred_element_type=jnp.float32) * (1.0 / L)
  y = jnp.dot(pooled, wt_ref[...], preferred_element_type=jnp.float32)
  y = y + b_ref[...]
  out_ref[...] = 0.5 * y * (1.0 + lax.erf(y * (2.0 ** -0.5)))


_mlp = pl.pallas_call(
    _mlp_body,
    grid=(B // BR,),
    in_specs=[
        pl.BlockSpec((BR, V), lambda i: (i, 0)),
        pl.BlockSpec((V, D), lambda i: (0, 0)),
        pl.BlockSpec((D, D), lambda i: (0, 0)),
        pl.BlockSpec((1, D), lambda i: (0, 0)),
    ],
    out_specs=pl.BlockSpec((BR, D), lambda i: (i, 0)),
    out_shape=jax.ShapeDtypeStruct((B, D), jnp.float32),
)


def kernel(ids, char_embed, W, b):
  ids_flat = ids.astype(jnp.int32).reshape(B * L)
  counts = _hist(ids_flat).reshape(B, V)
  return _mlp(counts, char_embed, W.T, b.reshape(1, D))
